# baseline (speedup 1.0000x reference)
import functools

import jax
import jax.numpy as jnp
from jax import lax
from jax.experimental import pallas as pl
from jax.experimental.pallas import tpu as pltpu
from jax.experimental.pallas import tpu_sc as plsc

VOCAB = 100000
TOKEN_DIM = 128
FEAT_DIM = 512
FEAT_EMB_DIM = 128
B, L = 1024, 50
N_ROWS = B * L

NC = 2
NS = 16
NW = NC * NS
ROWS_PER_W = N_ROWS // NW
CHUNK = 128
N_FULL = ROWS_PER_W // CHUNK
REM = ROWS_PER_W - N_FULL * CHUNK


def _sc_gather(table, idx_flat):
    mesh = plsc.VectorSubcoreMesh(core_axis_name="c", subcore_axis_name="s")

    @functools.partial(
        pl.kernel,
        mesh=mesh,
        out_type=jax.ShapeDtypeStruct((N_ROWS, TOKEN_DIM), jnp.float32),
        scratch_types=[
            pltpu.VMEM((ROWS_PER_W,), jnp.int32),
            pltpu.VMEM((CHUNK, TOKEN_DIM), jnp.float32),
            pltpu.VMEM((CHUNK, TOKEN_DIM), jnp.float32),
            pltpu.SemaphoreType.DMA,
            pltpu.SemaphoreType.DMA,
        ],
    )
    def gather_k(table_hbm, idx_hbm, out_hbm, idx_v, rows0, rows1, sem0, sem1):
        wid = lax.axis_index("s") * NC + lax.axis_index("c")
        base = wid * ROWS_PER_W
        pltpu.sync_copy(idx_hbm.at[pl.ds(base, ROWS_PER_W)], idx_v)

        bufs = (rows0, rows1)
        sems = (sem0, sem1)
        sizes = [CHUNK] * N_FULL + ([REM] if REM else [])
        copies = []
        for c, sz in enumerate(sizes):
            off = c * CHUNK
            buf = bufs[c % 2]
            cp = pltpu.make_async_copy(
                table_hbm.at[idx_v.at[pl.ds(off, sz)]],
                buf.at[pl.ds(0, sz)],
                sems[c % 2],
            )
            cp.start()
            copies.append((cp, off, sz, buf))
            if c >= 1:
                pcp, poff, psz, pbuf = copies[c - 1]
                pcp.wait()
                pltpu.sync_copy(
                    pbuf.at[pl.ds(0, psz)],
                    out_hbm.at[pl.ds(base + poff, psz)],
                )
        lcp, loff, lsz, lbuf = copies[-1]
        lcp.wait()
        pltpu.sync_copy(
            lbuf.at[pl.ds(0, lsz)],
            out_hbm.at[pl.ds(base + loff, lsz)],
        )

    return gather_k(table, idx_flat)


ROW_BLK = 512


def _tc_body(g_ref, f_ref, w_ref, b_ref, o_ref):
    o_ref[:, :TOKEN_DIM] = g_ref[...]
    acc = lax.dot_general(
        f_ref[...],
        w_ref[...],
        (((1,), (1,)), ((), ())),
        preferred_element_type=jnp.float32,
    )
    o_ref[:, TOKEN_DIM:] = acc + b_ref[...]


def _tc_project_concat(gathered, feat_flat, W, b2d):
    n_blk = N_ROWS // ROW_BLK
    return pl.pallas_call(
        _tc_body,
        grid=(n_blk,),
        in_specs=[
            pl.BlockSpec((ROW_BLK, TOKEN_DIM), lambda i: (i, 0)),
            pl.BlockSpec((ROW_BLK, FEAT_DIM), lambda i: (i, 0)),
            pl.BlockSpec((FEAT_EMB_DIM, FEAT_DIM), lambda i: (0, 0)),
            pl.BlockSpec((1, FEAT_EMB_DIM), lambda i: (0, 0)),
        ],
        out_specs=pl.BlockSpec((ROW_BLK, TOKEN_DIM + FEAT_EMB_DIM), lambda i: (i, 0)),
        out_shape=jax.ShapeDtypeStruct(
            (N_ROWS, TOKEN_DIM + FEAT_EMB_DIM), jnp.float32
        ),
    )(gathered, feat_flat, W, b2d)


@jax.jit
def kernel(tokens, features, table, W, b):
    idx_flat = tokens.reshape(N_ROWS).astype(jnp.int32)
    gathered = _sc_gather(table, idx_flat)
    feat_flat = features.reshape(N_ROWS, FEAT_DIM)
    out = _tc_project_concat(gathered, feat_flat, W, b.reshape(1, FEAT_EMB_DIM))
    return out.reshape(B, L, TOKEN_DIM + FEAT_EMB_DIM)
